# baseline (speedup 1.0000x reference)
import functools

import numpy as np
import jax
import jax.numpy as jnp
from jax import lax
from jax.experimental import pallas as pl
from jax.experimental.pallas import tpu as pltpu
from jax.experimental.pallas import tpu_sc as plsc

CHUNK = 128


def _sc_gather(item_table, user_table, e_idx, o_idx, user_idx, tgt_idx):
    n_pk = e_idx.shape[0]
    n_b = user_idx.shape[0]
    e = item_table.shape[1]

    info = plsc.get_sparse_core_info()
    nc, ns = info.num_cores, info.num_subcores
    nw = nc * ns

    gsz = CHUNK // 2
    ppw = n_pk // nw
    n_chunks = ppw // gsz
    bpw = n_b // nw

    e_idx2 = e_idx.reshape(nw, n_chunks, gsz)
    o_idx2 = o_idx.reshape(nw, n_chunks, gsz)
    user_idx2 = user_idx.reshape(nw, 1, bpw)
    tgt_idx2 = tgt_idx.reshape(nw, 1, bpw)

    mesh = plsc.VectorSubcoreMesh(core_axis_name="c", subcore_axis_name="s")

    @functools.partial(
        pl.kernel,
        mesh=mesh,
        compiler_params=pltpu.CompilerParams(use_tc_tiling_on_sc=False),
        out_type=[
            jax.ShapeDtypeStruct((n_pk, 2 * e), jnp.float32),
            jax.ShapeDtypeStruct((n_b, e), jnp.float32),
            jax.ShapeDtypeStruct((n_b, e), jnp.float32),
        ],
        scratch_types=[
            pltpu.VMEM((n_chunks, gsz), jnp.int32),
            pltpu.VMEM((n_chunks, gsz), jnp.int32),
            pltpu.VMEM((gsz, e), jnp.float32),
            pltpu.VMEM((gsz, e), jnp.float32),
            pltpu.VMEM((bpw, e), jnp.float32),
            pltpu.VMEM((1, bpw), jnp.int32),
            pltpu.SemaphoreType.DMA,
        ],
    )
    def gather_kernel(item_tab, user_tab, he_idx, ho_idx, u_idx, t_idx,
                      hist_out, user_out, tgt_out,
                      idx_e, idx_o, rows_e, rows_o, rows_b, idx_small, sem):
        wid = lax.axis_index("s") * nc + lax.axis_index("c")
        pltpu.sync_copy(he_idx.at[wid], idx_e)
        pltpu.sync_copy(ho_idx.at[wid], idx_o)
        pbase = wid * ppw

        def body(j, carry):
            pltpu.async_copy(item_tab.at[idx_e.at[j]], rows_e, sem).wait()
            pltpu.async_copy(item_tab.at[idx_o.at[j]], rows_o, sem).wait()
            row0 = pbase + j * gsz
            pltpu.sync_copy(rows_e,
                            hist_out.at[pl.ds(row0, gsz), pl.ds(0, e)])
            pltpu.sync_copy(rows_o,
                            hist_out.at[pl.ds(row0, gsz), pl.ds(e, e)])
            return carry

        lax.fori_loop(0, n_chunks, body, 0)

        base = wid * bpw
        pltpu.sync_copy(u_idx.at[wid], idx_small)
        pltpu.async_copy(user_tab.at[idx_small.at[0]], rows_b, sem).wait()
        pltpu.sync_copy(rows_b, user_out.at[pl.ds(base, bpw)])
        pltpu.sync_copy(t_idx.at[wid], idx_small)
        pltpu.async_copy(item_tab.at[idx_small.at[0]], rows_b, sem).wait()
        pltpu.sync_copy(rows_b, tgt_out.at[pl.ds(base, bpw)])

    return gather_kernel(item_table, user_table, e_idx2, o_idx2,
                         user_idx2, tgt_idx2)


def _tc_body(bt, t, e,
             hist_ref, te_ref, ue_ref, hl_ref, uf_ref, if_ref,
             wuf_ref, buf_ref, wif_ref, bif_ref,
             wa1_ref, ba1_ref, wa2_ref, ba2_ref, wa3_ref, ba3_ref,
             wu1_ref, bu1_ref, wu2_ref, bu2_ref, wu3_ref,
             wi1_ref, bi1_ref, wi2_ref, bi2_ref, wi3_ref,
             out_ref):
    f32 = jnp.float32
    th = t // 2
    q = te_ref[...]
    xp = hist_ref[...]
    wa1 = wa1_ref[...]
    a_blk = wa1[0:e]
    b_blk = wa1[e:2 * e]
    c_blk = wa1[2 * e:3 * e]
    d_blk = wa1[3 * e:4 * e]

    ze = jnp.zeros((e, e), f32)
    bc = b_blk - c_blk
    w_top = jnp.concatenate([jnp.concatenate([bc, ze], 1),
                             jnp.concatenate([ze, bc], 1)], 0)
    w_bot = jnp.concatenate([jnp.concatenate([d_blk, ze], 1),
                             jnp.concatenate([ze, d_blk], 1)], 0)

    q2 = jnp.concatenate([q, q], axis=1)
    xp3 = xp.reshape(bt, th, 2 * e)
    prodp = xp3 * q2[:, None, :]

    y = (jnp.dot(xp, w_top, preferred_element_type=f32)
         + jnp.dot(prodp.reshape(bt * th, 2 * e), w_bot,
                   preferred_element_type=f32))
    qpart = jnp.dot(q, a_blk + c_blk, preferred_element_type=f32) + ba1_ref[...]
    qp2 = jnp.concatenate([qpart, qpart], axis=1)
    h1 = jax.nn.sigmoid(y.reshape(bt, th, 2 * e) + qp2[:, None, :])

    wa2 = wa2_ref[...]
    nh = wa2.shape[1]
    z16 = jnp.zeros((e, nh), f32)
    w22 = jnp.concatenate([jnp.concatenate([wa2, z16], 1),
                           jnp.concatenate([z16, wa2], 1)], 0)
    ba2 = ba2_ref[...]
    ba22 = jnp.concatenate([ba2, ba2], axis=1)
    h2 = jax.nn.sigmoid(
        jnp.dot(h1.reshape(bt * th, 2 * e), w22, preferred_element_type=f32)
        + ba22)
    h23 = h2.reshape(bt, th, 2 * nh)
    wa3 = wa3_ref[...]
    z1 = jnp.zeros((1, nh), f32)
    wa3e = jnp.concatenate([wa3, z1], 1)[None, :, :]
    wa3o = jnp.concatenate([z1, wa3], 1)[None, :, :]
    se = jnp.sum(h23 * wa3e, axis=-1)
    so = jnp.sum(h23 * wa3o, axis=-1)
    score = jnp.concatenate([se, so], axis=1) + ba3_ref[0, 0]

    hl = hl_ref[...]
    tmask = lax.broadcasted_iota(jnp.int32, (bt, t), 1) < hl
    score = jnp.where(tmask, score, -1e9)
    m = jnp.max(score, axis=1, keepdims=True)
    ex = jnp.exp(score - m)
    attn = ex / jnp.sum(ex, axis=1, keepdims=True)
    ae = attn[:, :th, None]
    ao = attn[:, th:, None]
    a2 = jnp.concatenate([jnp.broadcast_to(ae, (bt, th, e)),
                          jnp.broadcast_to(ao, (bt, th, e))], axis=-1)
    hp128 = jnp.sum(xp3 * a2, axis=1)
    history = hp128[:, :e] + hp128[:, e:]

    user_feat = jax.nn.sigmoid(
        jnp.dot(uf_ref[...], wuf_ref[...], preferred_element_type=f32) + buf_ref[...])
    item_feat = jax.nn.sigmoid(
        jnp.dot(if_ref[...], wif_ref[...], preferred_element_type=f32) + bif_ref[...])

    cu = jnp.concatenate([ue_ref[...], history, user_feat], axis=1)
    u = jax.nn.relu(jnp.dot(cu, wu1_ref[...], preferred_element_type=f32) + bu1_ref[...])
    u = jax.nn.relu(jnp.dot(u, wu2_ref[...], preferred_element_type=f32) + bu2_ref[...])
    u = jax.nn.relu(jnp.dot(u, wu3_ref[...], preferred_element_type=f32))

    ci = jnp.concatenate([q, item_feat], axis=1)
    it = jax.nn.relu(jnp.dot(ci, wi1_ref[...], preferred_element_type=f32) + bi1_ref[...])
    it = jax.nn.relu(jnp.dot(it, wi2_ref[...], preferred_element_type=f32) + bi2_ref[...])
    it = jax.nn.relu(jnp.dot(it, wi3_ref[...], preferred_element_type=f32))

    out_ref[...] = jnp.sum(u * it, axis=1, keepdims=True)


def _tc_fused(hist_pk, tgt_emb, user_emb, history_len,
              user_features, item_features, p, bt):
    b, e = tgt_emb.shape
    t = hist_pk.shape[0] * 128 // (b * e)
    th = t // 2
    grid = (b // bt,)

    def full(shape):
        return pl.BlockSpec(shape, lambda i: (0,) * len(shape))

    in_specs = [
        pl.BlockSpec((bt * th, 128), lambda i: (i, 0)),
        pl.BlockSpec((bt, e), lambda i: (i, 0)),
        pl.BlockSpec((bt, e), lambda i: (i, 0)),
        pl.BlockSpec((bt, 1), lambda i: (i, 0)),
        pl.BlockSpec((bt, p['W_uf'].shape[0]), lambda i: (i, 0)),
        pl.BlockSpec((bt, p['W_if'].shape[0]), lambda i: (i, 0)),
        full(p['W_uf'].shape), full((1, e)),
        full(p['W_if'].shape), full((1, e)),
        full(p['Wa1'].shape), full((1, 64)),
        full(p['Wa2'].shape), full((1, 16)),
        full((1, 16)), full((1, 1)),
        full(p['Wu1'].shape), full((1, 200)),
        full(p['Wu2'].shape), full((1, 80)),
        full(p['Wu3'].shape),
        full(p['Wi1'].shape), full((1, 200)),
        full(p['Wi2'].shape), full((1, 80)),
        full(p['Wi3'].shape),
    ]
    out_spec = pl.BlockSpec((bt, 1), lambda i: (i, 0))

    body = functools.partial(_tc_body, bt, t, e)
    return pl.pallas_call(
        body,
        grid=grid,
        in_specs=in_specs,
        out_specs=out_spec,
        out_shape=jax.ShapeDtypeStruct((b, 1), jnp.float32),
    )(
        hist_pk, tgt_emb, user_emb, history_len.reshape(b, 1).astype(jnp.int32),
        user_features, item_features,
        p['W_uf'], p['b_uf'].reshape(1, -1),
        p['W_if'], p['b_if'].reshape(1, -1),
        p['Wa1'], p['ba1'].reshape(1, -1),
        p['Wa2'], p['ba2'].reshape(1, -1),
        p['Wa3'].reshape(1, -1), p['ba3'].reshape(1, 1),
        p['Wu1'], p['bu1'].reshape(1, -1),
        p['Wu2'], p['bu2'].reshape(1, -1),
        p['Wu3'],
        p['Wi1'], p['bi1'].reshape(1, -1),
        p['Wi2'], p['bi2'].reshape(1, -1),
        p['Wi3'],
    )


def kernel(user_id, target_item_id, history_item_id, history_len,
           user_features, item_features, params):
    p = params
    b, t = history_item_id.shape
    uid = user_id.reshape(b).astype(jnp.int32)
    tid = target_item_id.reshape(b).astype(jnp.int32)
    hid = history_item_id.astype(jnp.int32)
    e_idx = hid[:, :t // 2].reshape(b * t // 2)
    o_idx = hid[:, t // 2:].reshape(b * t // 2)

    hist_pk, user_emb, tgt_emb = _sc_gather(
        p['item_table'], p['user_table'], e_idx, o_idx, uid, tid)

    return _tc_fused(hist_pk, tgt_emb, user_emb, history_len,
                     user_features, item_features, p, bt=128)
